# baseline (speedup 1.0000x reference)
import functools

import jax
import jax.numpy as jnp
from jax import lax
from jax.experimental import pallas as pl
from jax.experimental.pallas import tpu as pltpu
from jax.experimental.pallas import tpu_sc as plsc

DIM = 128
B = 4096 * 50
NW = 32
BPW = B // NW
CHUNK = 128
NCH = BPW // CHUNK
NBUF = 6
LANES = 16

_mesh = plsc.VectorSubcoreMesh(core_axis_name="c", subcore_axis_name="s")


@functools.partial(
    pl.kernel,
    out_type=jax.ShapeDtypeStruct((B, DIM), jnp.float32),
    mesh=_mesh,
    scratch_types=(
        [pltpu.VMEM((CHUNK, DIM), jnp.float32) for _ in range(NBUF)]
        + [pltpu.VMEM((NBUF * CHUNK,), jnp.int32),
           pltpu.VMEM((NBUF * CHUNK,), jnp.int32)]
        + [pltpu.SemaphoreType.DMA for _ in range(2 * NBUF)]
    ),
    compiler_params=pltpu.CompilerParams(needs_layout_passes=False),
)
def _onehot_sc(x_hbm, table_hbm, out_hbm, *refs):
    rows = refs[:NBUF]
    nidx, ocol = refs[NBUF], refs[NBUF + 1]
    isem = refs[NBUF + 2:2 * NBUF + 2]
    ssem = refs[2 * NBUF + 2:3 * NBUF + 2]
    wid = lax.axis_index("s") * 2 + lax.axis_index("c")
    base = wid * BPW

    ones_v = jnp.full((LANES,), 1.0, jnp.float32)
    zeros_v = jnp.full((LANES,), 0.0, jnp.float32)
    lane = lax.iota(jnp.int32, LANES)

    def _start_idx(h, b):
        pltpu.async_copy(x_hbm.at[pl.ds(base + h * CHUNK, CHUNK)],
                         nidx.at[pl.ds(b * CHUNK, CHUNK)], isem[b])

    def _wait_idx(h, b):
        pltpu.make_async_copy(x_hbm.at[pl.ds(base + h * CHUNK, CHUNK)],
                              nidx.at[pl.ds(b * CHUNK, CHUNK)],
                              isem[b]).wait()

    def _start_store(h, b):
        pltpu.async_copy(rows[b],
                         out_hbm.at[pl.ds(base + h * CHUNK, CHUNK)], ssem[b])

    def _wait_store(h, b):
        pltpu.make_async_copy(rows[b],
                              out_hbm.at[pl.ds(base + h * CHUNK, CHUNK)],
                              ssem[b]).wait()

    _start_idx(0, 0)
    _start_idx(1, 1)

    def body(t, carry):
        for p in range(NBUF):
            h = NBUF * t + p

            @pl.when(h < NCH)
            def _():
                @pl.when(h + 2 < NCH)
                def _():
                    _start_idx(h + 2, (p + 2) % NBUF)

                _wait_idx(h, p)

                @pl.when(h < NBUF)
                def _():
                    def zbody(r, c):
                        for u in range(DIM // LANES):
                            rows[p][r, pl.ds(u * LANES, LANES)] = zeros_v
                        return c
                    lax.fori_loop(0, CHUNK, zbody, 0)

                @pl.when(h >= NBUF)
                def _():
                    _wait_store(h - NBUF, p)
                    for j in range(CHUNK // LANES):
                        sl = pl.ds(p * CHUNK + j * LANES, LANES)
                        rid = lane + (j * LANES)
                        plsc.store_scatter(rows[p], [rid, ocol[sl]], zeros_v)

                for j in range(CHUNK // LANES):
                    sl = pl.ds(p * CHUNK + j * LANES, LANES)
                    rid = lane + (j * LANES)
                    col = nidx[sl]
                    plsc.store_scatter(rows[p], [rid, col], ones_v)
                    ocol[sl] = col

                _start_store(h, p)
        return carry

    lax.fori_loop(0, (NCH + NBUF - 1) // NBUF, body, 0)

    for q in range(NBUF):
        h = NCH - NBUF + q
        _wait_store(h, h % NBUF)


def kernel(x, I):
    out = _onehot_sc(x.reshape(-1), I)
    return out.reshape(x.shape + (DIM,))


# speedup vs baseline: 2.0303x; 2.0303x over previous
import functools

import jax
import jax.numpy as jnp
from jax import lax
from jax.experimental import pallas as pl
from jax.experimental.pallas import tpu as pltpu
from jax.experimental.pallas import tpu_sc as plsc

DIM = 128
NSLAB = 4096
ROWS = 50
NW = 32
SPW = NSLAB // NW
IPW = SPW * ROWS
NBUF = 8
LANES = 16
NG = 4

_mesh = plsc.VectorSubcoreMesh(core_axis_name="c", subcore_axis_name="s")


@functools.partial(
    pl.kernel,
    out_type=jax.ShapeDtypeStruct((NSLAB, ROWS, DIM), jnp.float32),
    mesh=_mesh,
    scratch_types=(
        [pltpu.VMEM((ROWS, DIM), jnp.float32) for _ in range(NBUF)]
        + [pltpu.VMEM((IPW + 64,), jnp.int32),
           pltpu.VMEM((NBUF * 64,), jnp.int32)]
        + [pltpu.SemaphoreType.DMA, pltpu.SemaphoreType.DMA]
        + [pltpu.SemaphoreType.DMA for _ in range(NBUF)]
    ),
    compiler_params=pltpu.CompilerParams(needs_layout_passes=False),
)
def _onehot_sc(x_hbm, table_hbm, out_hbm, *refs):
    rows = refs[:NBUF]
    nidx, ocol = refs[NBUF], refs[NBUF + 1]
    isem = refs[NBUF + 2]
    ssem = refs[NBUF + 4:2 * NBUF + 4]
    wid = lax.axis_index("s") * 2 + lax.axis_index("c")
    sbase = wid * SPW

    ones_v = jnp.full((LANES,), 1.0, jnp.float32)
    zeros_v = jnp.full((LANES,), 0.0, jnp.float32)
    lane = lax.iota(jnp.int32, LANES)
    tail_m = lane < (ROWS - (NG - 1) * LANES)

    pltpu.async_copy(x_hbm.at[pl.ds(wid * IPW, IPW)],
                     nidx.at[pl.ds(0, IPW)], isem)
    pltpu.make_async_copy(x_hbm.at[pl.ds(wid * IPW, IPW)],
                          nidx.at[pl.ds(0, IPW)], isem).wait()

    def _start_store(h, b):
        pltpu.async_copy(rows[b], out_hbm.at[sbase + h], ssem[b])

    def _wait_store(h, b):
        pltpu.make_async_copy(rows[b], out_hbm.at[sbase + h],
                              ssem[b]).wait()

    def body(t, carry):
        for p in range(NBUF):
            h = NBUF * t + p

            @pl.when(h < NBUF)
            def _():
                def zbody(r, c):
                    for u in range(DIM // LANES):
                        rows[p][r, pl.ds(u * LANES, LANES)] = zeros_v
                    return c
                lax.fori_loop(0, ROWS, zbody, 0)

            @pl.when(h >= NBUF)
            def _():
                _wait_store(h - NBUF, p)
                for g in range(NG):
                    m = tail_m if g == NG - 1 else None
                    plsc.store_scatter(
                        rows[p],
                        [lane + g * LANES,
                         ocol[pl.ds(p * 64 + g * LANES, LANES)]],
                        zeros_v, mask=m)

            for g in range(NG):
                col = nidx[pl.ds(h * ROWS + g * LANES, LANES)]
                m = tail_m if g == NG - 1 else None
                plsc.store_scatter(rows[p], [lane + g * LANES, col],
                                   ones_v, mask=m)
                ocol[pl.ds(p * 64 + g * LANES, LANES)] = col

            _start_store(h, p)
        return carry

    lax.fori_loop(0, SPW // NBUF, body, 0)

    for q in range(NBUF):
        _wait_store(SPW - NBUF + q, q)


def kernel(x, I):
    return _onehot_sc(x.reshape(-1), I)
